# baseline (speedup 1.0000x reference)
import functools
import math

import jax
import jax.numpy as jnp
from jax import lax
from jax.experimental import pallas as pl
from jax.experimental.pallas import tpu as pltpu
from jax.experimental.pallas import tpu_sc as plsc

_SCALE = math.sqrt(128.0)


def _make_sc_gather(V, D, B):
    info = plsc.get_sparse_core_info()
    NC, NS = info.num_cores, info.num_subcores
    NW = NC * NS
    assert B % (8 * NW) == 0
    b_per_w = B // NW
    mesh = plsc.VectorSubcoreMesh(core_axis_name="c", subcore_axis_name="s")

    @functools.partial(
        pl.kernel,
        mesh=mesh,
        out_type=jax.ShapeDtypeStruct((B, D), jnp.float32),
        scratch_types=[
            pltpu.VMEM((b_per_w,), jnp.int32),
            pltpu.VMEM((b_per_w, D), jnp.float32),
            pltpu.SemaphoreType.DMA,
        ],
    )
    def sc_gather(table_hbm, idx_hbm, out_hbm, idx_v, rows_v, sem):
        wid = lax.axis_index("s") * NC + lax.axis_index("c")
        base = wid * b_per_w
        pltpu.sync_copy(idx_hbm.at[pl.ds(base, b_per_w)], idx_v)
        pltpu.async_copy(table_hbm.at[idx_v], rows_v, sem).wait()
        pltpu.sync_copy(rows_v, out_hbm.at[pl.ds(base, b_per_w)])

    return sc_gather


_BB = 128
_NBUF = 6


def _mul_body(node_hbm, r_hbm, out_hbm, in_buf, out_buf, r_v, in_sem, out_sem, r_sem):
    B, H, E = node_hbm.shape
    nblk = B // _BB

    pltpu.make_async_copy(r_hbm, r_v, r_sem).start()
    for s in range(_NBUF):
        pltpu.make_async_copy(
            node_hbm.at[pl.ds(s * _BB, _BB)], in_buf.at[s], in_sem.at[s]
        ).start()
    pltpu.make_async_copy(r_hbm, r_v, r_sem).wait()
    r_v[...] = r_v[...] * _SCALE

    def step(i, carry):
        slot = lax.rem(i, _NBUF)
        pltpu.make_async_copy(
            node_hbm.at[pl.ds(i * _BB, _BB)], in_buf.at[slot], in_sem.at[slot]
        ).wait()

        @pl.when(i >= _NBUF)
        def _drain():
            pltpu.make_async_copy(
                out_buf.at[slot],
                out_hbm.at[pl.ds((i - _NBUF) * _BB, _BB)],
                out_sem.at[slot],
            ).wait()

        def sub(j, c):
            rs = r_v[pl.ds(i * _BB + j * 8, 8), :][:, None, :]
            out_buf[slot, pl.ds(j * 8, 8)] = in_buf[slot, pl.ds(j * 8, 8)] * rs
            return c

        lax.fori_loop(0, _BB // 8, sub, 0)

        pltpu.make_async_copy(
            out_buf.at[slot], out_hbm.at[pl.ds(i * _BB, _BB)], out_sem.at[slot]
        ).start()

        @pl.when(i + _NBUF < nblk)
        def _refill():
            pltpu.make_async_copy(
                node_hbm.at[pl.ds((i + _NBUF) * _BB, _BB)],
                in_buf.at[slot],
                in_sem.at[slot],
            ).start()

        return carry

    lax.fori_loop(0, nblk, step, 0)

    def drain(i, carry):
        slot = lax.rem(nblk - _NBUF + i, _NBUF)
        pltpu.make_async_copy(
            out_buf.at[slot],
            out_hbm.at[pl.ds((nblk - _NBUF + i) * _BB, _BB)],
            out_sem.at[slot],
        ).wait()
        return carry

    lax.fori_loop(0, _NBUF, drain, 0)


def kernel(node_emb, relation, rela_emb):
    B, H, E = node_emb.shape
    V = rela_emb.shape[0]

    r_emb = _make_sc_gather(V, E, B)(rela_emb, relation)

    BB = 256
    out = pl.pallas_call(
        _grid_mul_body,
        grid=(B // BB,),
        in_specs=[
            pl.BlockSpec((BB, H, E), lambda i: (i, 0, 0)),
            pl.BlockSpec((BB, E), lambda i: (i, 0)),
        ],
        out_specs=pl.BlockSpec((BB, H, E), lambda i: (i, 0, 0)),
        out_shape=jax.ShapeDtypeStruct((B, H, E), jnp.float32),
        compiler_params=pltpu.CompilerParams(
            dimension_semantics=("parallel",),
        ),
    )(node_emb, r_emb)
    return out


def _grid_mul_body(node_ref, r_ref, out_ref):
    out_ref[...] = node_ref[...] * (r_ref[...] * _SCALE)[:, None, :]


# speedup vs baseline: 2.5898x; 2.5898x over previous
import functools
import math

import jax
import jax.numpy as jnp
from jax import lax
from jax.experimental import pallas as pl
from jax.experimental.pallas import tpu as pltpu
from jax.experimental.pallas import tpu_sc as plsc

_SCALE = math.sqrt(128.0)


def _make_sc_gather(V, D, B):
    info = plsc.get_sparse_core_info()
    NC, NS = info.num_cores, info.num_subcores
    NW = NC * NS
    assert B % (8 * NW) == 0
    b_per_w = B // NW
    mesh = plsc.VectorSubcoreMesh(core_axis_name="c", subcore_axis_name="s")

    @functools.partial(
        pl.kernel,
        mesh=mesh,
        out_type=jax.ShapeDtypeStruct((B, D), jnp.float32),
        scratch_types=[
            pltpu.VMEM((b_per_w,), jnp.int32),
            pltpu.VMEM((b_per_w, D), jnp.float32),
            pltpu.SemaphoreType.DMA,
        ],
    )
    def sc_gather(table_hbm, idx_hbm, out_hbm, idx_v, rows_v, sem):
        wid = lax.axis_index("s") * NC + lax.axis_index("c")
        base = wid * b_per_w
        pltpu.sync_copy(idx_hbm.at[pl.ds(base, b_per_w)], idx_v)
        pltpu.async_copy(table_hbm.at[idx_v], rows_v, sem).wait()
        pltpu.sync_copy(rows_v, out_hbm.at[pl.ds(base, b_per_w)])

    return sc_gather


_BB = 128
_NBUF = 6


def _mul_body(node_hbm, r_hbm, out_hbm, in_buf, out_buf, r_v, in_sem, out_sem, r_sem):
    B, H, E = node_hbm.shape
    nblk = B // _BB

    pltpu.make_async_copy(r_hbm, r_v, r_sem).start()
    for s in range(_NBUF):
        pltpu.make_async_copy(
            node_hbm.at[pl.ds(s * _BB, _BB)], in_buf.at[s], in_sem.at[s]
        ).start()
    pltpu.make_async_copy(r_hbm, r_v, r_sem).wait()
    r_v[...] = r_v[...] * _SCALE

    def step(i, carry):
        slot = lax.rem(i, _NBUF)
        pltpu.make_async_copy(
            node_hbm.at[pl.ds(i * _BB, _BB)], in_buf.at[slot], in_sem.at[slot]
        ).wait()

        @pl.when(i >= _NBUF)
        def _drain():
            pltpu.make_async_copy(
                out_buf.at[slot],
                out_hbm.at[pl.ds((i - _NBUF) * _BB, _BB)],
                out_sem.at[slot],
            ).wait()

        def sub(j, c):
            rs = r_v[pl.ds(i * _BB + j * 8, 8), :][:, None, :]
            out_buf[slot, pl.ds(j * 8, 8)] = in_buf[slot, pl.ds(j * 8, 8)] * rs
            return c

        lax.fori_loop(0, _BB // 8, sub, 0)

        pltpu.make_async_copy(
            out_buf.at[slot], out_hbm.at[pl.ds(i * _BB, _BB)], out_sem.at[slot]
        ).start()

        @pl.when(i + _NBUF < nblk)
        def _refill():
            pltpu.make_async_copy(
                node_hbm.at[pl.ds((i + _NBUF) * _BB, _BB)],
                in_buf.at[slot],
                in_sem.at[slot],
            ).start()

        return carry

    lax.fori_loop(0, nblk, step, 0)

    def drain(i, carry):
        slot = lax.rem(nblk - _NBUF + i, _NBUF)
        pltpu.make_async_copy(
            out_buf.at[slot],
            out_hbm.at[pl.ds((nblk - _NBUF + i) * _BB, _BB)],
            out_sem.at[slot],
        ).wait()
        return carry

    lax.fori_loop(0, _NBUF, drain, 0)


def kernel(node_emb, relation, rela_emb):
    B, H, E = node_emb.shape
    V = rela_emb.shape[0]

    r_emb = _make_sc_gather(V, E, B)(rela_emb, relation)

    nodeT = jnp.transpose(node_emb, (1, 0, 2))
    BB = 256
    outT = pl.pallas_call(
        _grid_mul_body,
        grid=(B // BB,),
        in_specs=[
            pl.BlockSpec((H, BB, E), lambda i: (0, i, 0)),
            pl.BlockSpec((BB, E), lambda i: (i, 0)),
        ],
        out_specs=pl.BlockSpec((H, BB, E), lambda i: (0, i, 0)),
        out_shape=jax.ShapeDtypeStruct((H, B, E), jnp.float32),
        compiler_params=pltpu.CompilerParams(
            dimension_semantics=("parallel",),
        ),
    )(nodeT, r_emb)
    return jnp.transpose(outT, (1, 0, 2))


def _grid_mul_body(node_ref, r_ref, out_ref):
    out_ref[...] = node_ref[...] * (r_ref[...] * _SCALE)[None, :, :]


# speedup vs baseline: 2.6380x; 1.0186x over previous
import functools
import math

import jax
import jax.numpy as jnp
from jax import lax
from jax.experimental import pallas as pl
from jax.experimental.pallas import tpu as pltpu
from jax.experimental.pallas import tpu_sc as plsc

_SCALE = math.sqrt(128.0)


def _make_sc_gather(V, D, B):
    info = plsc.get_sparse_core_info()
    NC, NS = info.num_cores, info.num_subcores
    NW = NC * NS
    assert B % (8 * NW) == 0
    b_per_w = B // NW
    mesh = plsc.VectorSubcoreMesh(core_axis_name="c", subcore_axis_name="s")

    @functools.partial(
        pl.kernel,
        mesh=mesh,
        out_type=jax.ShapeDtypeStruct((B, D), jnp.float32),
        scratch_types=[
            pltpu.VMEM((b_per_w,), jnp.int32),
            pltpu.VMEM((b_per_w, D), jnp.float32),
            pltpu.SemaphoreType.DMA,
        ],
    )
    def sc_gather(table_hbm, idx_hbm, out_hbm, idx_v, rows_v, sem):
        wid = lax.axis_index("s") * NC + lax.axis_index("c")
        base = wid * b_per_w
        pltpu.sync_copy(idx_hbm.at[pl.ds(base, b_per_w)], idx_v)
        pltpu.async_copy(table_hbm.at[idx_v], rows_v, sem).wait()
        pltpu.sync_copy(rows_v, out_hbm.at[pl.ds(base, b_per_w)])

    return sc_gather


_BB = 128
_NBUF = 6


def _mul_body(node_hbm, r_hbm, out_hbm, in_buf, out_buf, r_v, in_sem, out_sem, r_sem):
    B, H, E = node_hbm.shape
    nblk = B // _BB

    pltpu.make_async_copy(r_hbm, r_v, r_sem).start()
    for s in range(_NBUF):
        pltpu.make_async_copy(
            node_hbm.at[pl.ds(s * _BB, _BB)], in_buf.at[s], in_sem.at[s]
        ).start()
    pltpu.make_async_copy(r_hbm, r_v, r_sem).wait()
    r_v[...] = r_v[...] * _SCALE

    def step(i, carry):
        slot = lax.rem(i, _NBUF)
        pltpu.make_async_copy(
            node_hbm.at[pl.ds(i * _BB, _BB)], in_buf.at[slot], in_sem.at[slot]
        ).wait()

        @pl.when(i >= _NBUF)
        def _drain():
            pltpu.make_async_copy(
                out_buf.at[slot],
                out_hbm.at[pl.ds((i - _NBUF) * _BB, _BB)],
                out_sem.at[slot],
            ).wait()

        def sub(j, c):
            rs = r_v[pl.ds(i * _BB + j * 8, 8), :][:, None, :]
            out_buf[slot, pl.ds(j * 8, 8)] = in_buf[slot, pl.ds(j * 8, 8)] * rs
            return c

        lax.fori_loop(0, _BB // 8, sub, 0)

        pltpu.make_async_copy(
            out_buf.at[slot], out_hbm.at[pl.ds(i * _BB, _BB)], out_sem.at[slot]
        ).start()

        @pl.when(i + _NBUF < nblk)
        def _refill():
            pltpu.make_async_copy(
                node_hbm.at[pl.ds((i + _NBUF) * _BB, _BB)],
                in_buf.at[slot],
                in_sem.at[slot],
            ).start()

        return carry

    lax.fori_loop(0, nblk, step, 0)

    def drain(i, carry):
        slot = lax.rem(nblk - _NBUF + i, _NBUF)
        pltpu.make_async_copy(
            out_buf.at[slot],
            out_hbm.at[pl.ds((nblk - _NBUF + i) * _BB, _BB)],
            out_sem.at[slot],
        ).wait()
        return carry

    lax.fori_loop(0, _NBUF, drain, 0)


def kernel(node_emb, relation, rela_emb):
    B, H, E = node_emb.shape
    V = rela_emb.shape[0]

    r_emb = _make_sc_gather(V, E, B)(rela_emb, relation)

    nodeT = jnp.transpose(node_emb, (1, 0, 2))
    BB = 512
    outT = pl.pallas_call(
        _grid_mul_body,
        grid=(B // BB,),
        in_specs=[
            pl.BlockSpec((H, BB, E), lambda i: (0, i, 0)),
            pl.BlockSpec((BB, E), lambda i: (i, 0)),
        ],
        out_specs=pl.BlockSpec((H, BB, E), lambda i: (0, i, 0)),
        out_shape=jax.ShapeDtypeStruct((H, B, E), jnp.float32),
        compiler_params=pltpu.CompilerParams(
            dimension_semantics=("parallel",),
        ),
    )(nodeT, r_emb)
    return jnp.transpose(outT, (1, 0, 2))


def _grid_mul_body(node_ref, r_ref, out_ref):
    out_ref[...] = node_ref[...] * (r_ref[...] * _SCALE)[None, :, :]


# speedup vs baseline: 3.3816x; 1.2819x over previous
import functools
import math

import jax
import jax.numpy as jnp
from jax import lax
from jax.experimental import pallas as pl
from jax.experimental.pallas import tpu as pltpu
from jax.experimental.pallas import tpu_sc as plsc

_SCALE = math.sqrt(128.0)


def _make_sc_gather(V, D, B):
    info = plsc.get_sparse_core_info()
    NC, NS = info.num_cores, info.num_subcores
    NW = NC * NS
    assert B % (8 * NW) == 0
    b_per_w = B // NW
    mesh = plsc.VectorSubcoreMesh(core_axis_name="c", subcore_axis_name="s")

    @functools.partial(
        pl.kernel,
        mesh=mesh,
        out_type=jax.ShapeDtypeStruct((B, D), jnp.float32),
        scratch_types=[
            pltpu.VMEM((b_per_w,), jnp.int32),
            pltpu.VMEM((b_per_w, D), jnp.float32),
            pltpu.SemaphoreType.DMA,
        ],
    )
    def sc_gather(table_hbm, idx_hbm, out_hbm, idx_v, rows_v, sem):
        wid = lax.axis_index("s") * NC + lax.axis_index("c")
        base = wid * b_per_w
        pltpu.sync_copy(idx_hbm.at[pl.ds(base, b_per_w)], idx_v)
        pltpu.async_copy(table_hbm.at[idx_v], rows_v, sem).wait()
        pltpu.sync_copy(rows_v, out_hbm.at[pl.ds(base, b_per_w)])

    return sc_gather


_BB = 128
_NBUF = 6


def _mul_body(node_hbm, r_hbm, out_hbm, in_buf, out_buf, r_v, in_sem, out_sem, r_sem):
    B, H, E = node_hbm.shape
    nblk = B // _BB

    pltpu.make_async_copy(r_hbm, r_v, r_sem).start()
    for s in range(_NBUF):
        pltpu.make_async_copy(
            node_hbm.at[pl.ds(s * _BB, _BB)], in_buf.at[s], in_sem.at[s]
        ).start()
    pltpu.make_async_copy(r_hbm, r_v, r_sem).wait()
    r_v[...] = r_v[...] * _SCALE

    def step(i, carry):
        slot = lax.rem(i, _NBUF)
        pltpu.make_async_copy(
            node_hbm.at[pl.ds(i * _BB, _BB)], in_buf.at[slot], in_sem.at[slot]
        ).wait()

        @pl.when(i >= _NBUF)
        def _drain():
            pltpu.make_async_copy(
                out_buf.at[slot],
                out_hbm.at[pl.ds((i - _NBUF) * _BB, _BB)],
                out_sem.at[slot],
            ).wait()

        def sub(j, c):
            rs = r_v[pl.ds(i * _BB + j * 8, 8), :][:, None, :]
            out_buf[slot, pl.ds(j * 8, 8)] = in_buf[slot, pl.ds(j * 8, 8)] * rs
            return c

        lax.fori_loop(0, _BB // 8, sub, 0)

        pltpu.make_async_copy(
            out_buf.at[slot], out_hbm.at[pl.ds(i * _BB, _BB)], out_sem.at[slot]
        ).start()

        @pl.when(i + _NBUF < nblk)
        def _refill():
            pltpu.make_async_copy(
                node_hbm.at[pl.ds((i + _NBUF) * _BB, _BB)],
                in_buf.at[slot],
                in_sem.at[slot],
            ).start()

        return carry

    lax.fori_loop(0, nblk, step, 0)

    def drain(i, carry):
        slot = lax.rem(nblk - _NBUF + i, _NBUF)
        pltpu.make_async_copy(
            out_buf.at[slot],
            out_hbm.at[pl.ds((nblk - _NBUF + i) * _BB, _BB)],
            out_sem.at[slot],
        ).wait()
        return carry

    lax.fori_loop(0, _NBUF, drain, 0)


def kernel(node_emb, relation, rela_emb):
    B, H, E = node_emb.shape
    V = rela_emb.shape[0]

    nodeT = jnp.transpose(node_emb, (1, 0, 2))
    BB = 512
    Vp = 1024
    table = jnp.pad(rela_emb, ((0, Vp - V), (0, 0)))
    rel3 = relation.reshape(B // BB, 1, BB)
    outT = pl.pallas_call(
        _grid_mul_body,
        grid=(B // BB,),
        in_specs=[
            pl.BlockSpec((H, BB, E), lambda i: (0, i, 0)),
            pl.BlockSpec((1, 1, BB), lambda i: (i, 0, 0)),
            pl.BlockSpec((Vp, E), lambda i: (0, 0)),
        ],
        out_specs=pl.BlockSpec((H, BB, E), lambda i: (0, i, 0)),
        out_shape=jax.ShapeDtypeStruct((H, B, E), jnp.float32),
        compiler_params=pltpu.CompilerParams(
            dimension_semantics=("parallel",),
        ),
    )(nodeT, rel3, table)
    return jnp.transpose(outT, (1, 0, 2))


def _grid_mul_body(node_ref, rel_ref, table_ref, out_ref):
    BB = rel_ref.shape[-1]
    Vp = table_ref.shape[0]
    rel = rel_ref[0, 0, :]
    onehot = (
        jax.lax.broadcasted_iota(jnp.int32, (BB, Vp), 1) == rel[:, None]
    ).astype(jnp.float32)
    r = jnp.dot(onehot, table_ref[...], preferred_element_type=jnp.float32)
    out_ref[...] = node_ref[...] * (r * _SCALE)[None, :, :]
